# baseline (speedup 1.0000x reference)
import jax
import jax.numpy as jnp
from jax.experimental import pallas as pl
from jax.experimental.pallas import tpu as pltpu


def _clam_block(h_ref, W1_ref, b1_ref, Wa_ref, ba_ref, Wb_ref, bb_ref,
                wc_ref, bc_ref,
                A_ref, s_out_ref, acc_out_ref,
                m_ref, s_ref, acc_ref):
    j = pl.program_id(1)

    @pl.when(j == 0)
    def _init():
        m_ref[...] = (jnp.sum(jnp.abs(wc_ref[...]), axis=1, keepdims=True)
                      + jnp.abs(bc_ref[...]))
        s_ref[...] = jnp.zeros_like(s_ref)
        acc_ref[...] = jnp.zeros_like(acc_ref)

    h_blk = h_ref[...].astype(jnp.bfloat16)
    h1 = jnp.dot(h_blk, W1_ref[...], preferred_element_type=jnp.float32)
    h1 = jnp.maximum(h1 + b1_ref[...], 0.0)
    h1b = h1.astype(jnp.bfloat16)
    a = jnp.tanh(jnp.dot(h1b, Wa_ref[...], preferred_element_type=jnp.float32)
                 + ba_ref[...])
    b = jax.nn.sigmoid(jnp.dot(h1b, Wb_ref[...], preferred_element_type=jnp.float32)
                       + bb_ref[...])
    g = a * b
    att = jnp.sum(g * wc_ref[...], axis=1, keepdims=True) + bc_ref[...]
    A_ref[...] = att

    p = jnp.exp(att - m_ref[...])
    s_ref[...] = s_ref[...] + jnp.sum(p, axis=(0, 1), keepdims=True)
    pw = jax.lax.dot_general(p, h1, (((0,), (0,)), ((), ())),
                             preferred_element_type=jnp.float32)
    acc_ref[...] = acc_ref[...] + pw

    @pl.when(j == pl.num_programs(1) - 1)
    def _flush():
        s_out_ref[...] = s_ref[...][None]
        acc_out_ref[...] = acc_ref[...][None]


def _head(s_ref, acc_ref, Wcls_ref, bcls_ref,
          logits_ref, yprob_ref, yhat_ref):
    s = jnp.sum(s_ref[...], axis=0)
    acc = jnp.sum(acc_ref[...], axis=0)
    M = acc / s
    logits = jnp.dot(M, Wcls_ref[...], preferred_element_type=jnp.float32)
    logits = logits + bcls_ref[...]
    logits_ref[...] = logits
    mx = jnp.max(logits, axis=1, keepdims=True)
    e = jnp.exp(logits - mx)
    yprob_ref[...] = e / jnp.sum(e, axis=1, keepdims=True)
    c = logits.shape[1]
    idx = jax.lax.broadcasted_iota(jnp.int32, logits.shape, 1)
    yhat_ref[...] = jnp.min(jnp.where(logits == mx, idx, c), axis=1,
                            keepdims=True)


def kernel(h, W1, b1, Wa, ba, Wb, bb, Wc, bc, Wcls, bcls):
    n, d_in = h.shape
    d_hid = W1.shape[1]
    d_att = Wa.shape[1]
    n_classes = Wcls.shape[1]

    split = 2
    blk = 1000
    if n % (blk * split) != 0:
        blk = next(b for b in (500, 250, 200, 100, 50, 25, 10, 5, 2, 1)
                   if n % (b * split) == 0)
    nblkj = n // blk // split

    W1_b = W1.astype(jnp.bfloat16)
    Wa_b = Wa.astype(jnp.bfloat16)
    Wb_b = Wb.astype(jnp.bfloat16)
    b1_r = b1.reshape(1, d_hid)
    ba_r = ba.reshape(1, d_att)
    bb_r = bb.reshape(1, d_att)
    wc_r = Wc.reshape(1, d_att)
    bc_r = bc.reshape(1, 1)
    bcls_r = bcls.reshape(1, n_classes)

    const = lambda i, j: (0, 0)
    A_col, s_part, acc_part = pl.pallas_call(
        _clam_block,
        grid=(split, nblkj),
        in_specs=[
            pl.BlockSpec((blk, d_in), lambda i, j: (i * nblkj + j, 0)),
            pl.BlockSpec((d_in, d_hid), const),
            pl.BlockSpec((1, d_hid), const),
            pl.BlockSpec((d_hid, d_att), const),
            pl.BlockSpec((1, d_att), const),
            pl.BlockSpec((d_hid, d_att), const),
            pl.BlockSpec((1, d_att), const),
            pl.BlockSpec((1, d_att), const),
            pl.BlockSpec((1, 1), const),
        ],
        out_specs=[
            pl.BlockSpec((blk, 1), lambda i, j: (i * nblkj + j, 0)),
            pl.BlockSpec((1, 1, 1), lambda i, j: (i, 0, 0)),
            pl.BlockSpec((1, 1, d_hid), lambda i, j: (i, 0, 0)),
        ],
        out_shape=[
            jax.ShapeDtypeStruct((n, 1), jnp.float32),
            jax.ShapeDtypeStruct((split, 1, 1), jnp.float32),
            jax.ShapeDtypeStruct((split, 1, d_hid), jnp.float32),
        ],
        scratch_shapes=[
            pltpu.VMEM((1, 1), jnp.float32),
            pltpu.VMEM((1, 1), jnp.float32),
            pltpu.VMEM((1, d_hid), jnp.float32),
        ],
        compiler_params=pltpu.CompilerParams(
            dimension_semantics=("parallel", "arbitrary"),
        ),
    )(h, W1_b, b1_r, Wa_b, ba_r, Wb_b, bb_r, wc_r, bc_r)

    logits, y_prob, y_hat = pl.pallas_call(
        _head,
        in_specs=[
            pl.BlockSpec((split, 1, 1), lambda: (0, 0, 0)),
            pl.BlockSpec((split, 1, d_hid), lambda: (0, 0, 0)),
            pl.BlockSpec((d_hid, n_classes), lambda: (0, 0)),
            pl.BlockSpec((1, n_classes), lambda: (0, 0)),
        ],
        out_specs=[
            pl.BlockSpec((1, n_classes), lambda: (0, 0)),
            pl.BlockSpec((1, n_classes), lambda: (0, 0)),
            pl.BlockSpec((1, 1), lambda: (0, 0)),
        ],
        out_shape=[
            jax.ShapeDtypeStruct((1, n_classes), jnp.float32),
            jax.ShapeDtypeStruct((1, n_classes), jnp.float32),
            jax.ShapeDtypeStruct((1, 1), jnp.int32),
        ],
    )(s_part, acc_part, Wcls, bcls_r)

    return (logits, y_prob, y_hat, A_col.reshape(1, n))
